# baseline (speedup 1.0000x reference)
import functools

import jax
import jax.numpy as jnp
from jax import lax
from jax.experimental import pallas as pl
from jax.experimental.pallas import tpu as pltpu
from jax.experimental.pallas import tpu_sc as plsc

K = 64
N = 8192
L = 16
R_TOTAL = 8192
NW = 32
ROWS_PER_W = R_TOTAL // NW
BATCH = 4
NGRP = 32


def _rev(v):
    return lax.rev(v, (0,))


def _sort16(v, desc):
    if desc:
        return plsc.sort_key_val(v, v, descending=True)[0]
    return jnp.sort(v)


def _sort_bitonic(vs, desc):
    if len(vs) == 1:
        return [_sort16(vs[0], desc)]
    h = len(vs) // 2
    lo = [jnp.minimum(a, b) for a, b in zip(vs[:h], vs[h:])]
    hi = [jnp.maximum(a, b) for a, b in zip(vs[:h], vs[h:])]
    if desc:
        return _sort_bitonic(hi, True) + _sort_bitonic(lo, True)
    return _sort_bitonic(lo, False) + _sort_bitonic(hi, False)


def _merge(A, B, desc, cap=False):
    hi = [jnp.maximum(a, b) for a, b in zip(A, B)]
    if cap:
        return _sort_bitonic(hi, desc)
    lo = [jnp.minimum(a, b) for a, b in zip(A, B)]
    if desc:
        return _sort_bitonic(hi, True) + _sort_bitonic(lo, True)
    return _sort_bitonic(lo, False) + _sort_bitonic(hi, False)


def _kv_sort_bitonic(ks, vs, desc):
    if len(ks) == 1:
        sk, sv = plsc.sort_key_val(ks[0], vs[0], descending=desc)
        return [sk], [sv]
    h = len(ks) // 2
    m = [a <= b for a, b in zip(ks[:h], ks[h:])]
    lok = [jnp.minimum(a, b) for a, b in zip(ks[:h], ks[h:])]
    hik = [jnp.maximum(a, b) for a, b in zip(ks[:h], ks[h:])]
    lov = [jnp.where(mm, a, b) for mm, a, b in zip(m, vs[:h], vs[h:])]
    hiv = [jnp.where(mm, b, a) for mm, a, b in zip(m, vs[:h], vs[h:])]
    if desc:
        k1, v1 = _kv_sort_bitonic(hik, hiv, True)
        k2, v2 = _kv_sort_bitonic(lok, lov, True)
        return k1 + k2, v1 + v2
    k1, v1 = _kv_sort_bitonic(lok, lov, False)
    k2, v2 = _kv_sort_bitonic(hik, hiv, False)
    return k1 + k2, v1 + v2


def _kv_merge(Ak, Av, Bk, Bv, desc, cap=False):
    m = [a <= b for a, b in zip(Ak, Bk)]
    hik = [jnp.maximum(a, b) for a, b in zip(Ak, Bk)]
    hiv = [jnp.where(mm, b, a) for mm, a, b in zip(m, Av, Bv)]
    if cap:
        return _kv_sort_bitonic(hik, hiv, desc)
    lok = [jnp.minimum(a, b) for a, b in zip(Ak, Bk)]
    lov = [jnp.where(mm, a, b) for mm, a, b in zip(m, Av, Bv)]
    if desc:
        k1, v1 = _kv_sort_bitonic(hik, hiv, True)
        k2, v2 = _kv_sort_bitonic(lok, lov, True)
        return k1 + k2, v1 + v2
    k1, v1 = _kv_sort_bitonic(lok, lov, False)
    k2, v2 = _kv_sort_bitonic(hik, hiv, False)
    return k1 + k2, v1 + v2


def _process_row(row_v, out_v, off, r):
    iota = lax.iota(jnp.int32, L)
    lists = []
    for g in range(NGRP):
        vs = [row_v[pl.ds(off + g * 256 + j * L, L)] for j in range(16)]
        while len(vs) > 1:
            vs = [jnp.maximum(a, b) for a, b in zip(vs[0::2], vs[1::2])]
        sk, sv = plsc.sort_key_val(vs[0], g * 256 + iota,
                                   descending=(g % 2 == 1))
        lists.append(([sk], [sv]))
    while len(lists) > 1:
        nxt = []
        for i, ((ak, av), (bk, bv)) in enumerate(
                zip(lists[0::2], lists[1::2])):
            nxt.append(_kv_merge(ak, av, bk, bv, desc=(i % 2 == 1),
                                 cap=(len(ak) == 4)))
        lists = nxt
    vals4 = lists[0][1]
    leaves = []
    for idx, v in enumerate(vals4):
        base = v + off
        for j in range(16):
            i = idx * 16 + j
            leaves.append(_sort16(plsc.load_gather(row_v, [base + j * L]),
                                  desc=(i % 2 == 1)))
    ls = [[x] for x in leaves]
    while len(ls) > 1:
        ls = [_merge(a, b, desc=(i % 2 == 1), cap=(len(a) == 4))
              for i, (a, b) in enumerate(zip(ls[0::2], ls[1::2]))]
    top = ls[0]
    for j in range(4):
        out_v[pl.ds(r * K + j * L, L)] = _rev(top[3 - j])


def _sc_topk(x_hbm, out_hbm, row_a, row_b, out_v, sem_a, sem_b):
    wid = lax.axis_index("s") * 2 + lax.axis_index("c")
    base = wid * ROWS_PER_W
    nb = ROWS_PER_W // BATCH

    def start_batch(buf, sem, batch_idx):
        rows0 = base + batch_idx * BATCH
        bb = rows0 // 1024
        ss = rows0 % 1024
        for i in range(BATCH):
            pltpu.make_async_copy(
                x_hbm.at[bb, ss + i], buf.at[pl.ds(i * N, N)], sem).start()

    def wait_batch(buf, sem):
        for i in range(BATCH):
            pltpu.make_async_copy(
                x_hbm.at[0, 0], buf.at[pl.ds(i * N, N)], sem).wait()

    def process_batch(buf, batch_idx):
        def row_body(i, _):
            _process_row(buf, out_v, i * N, batch_idx * BATCH + i)
            return 0
        lax.fori_loop(0, BATCH, row_body, 0, unroll=False)

    start_batch(row_a, sem_a, 0)

    def body(b2, _):
        start_batch(row_b, sem_b, 2 * b2 + 1)
        wait_batch(row_a, sem_a)
        process_batch(row_a, 2 * b2)

        @pl.when(b2 < nb // 2 - 1)
        def _():
            start_batch(row_a, sem_a, 2 * b2 + 2)

        wait_batch(row_b, sem_b)
        process_batch(row_b, 2 * b2 + 1)
        return 0

    lax.fori_loop(0, nb // 2, body, 0, unroll=False)

    out_copy = pltpu.make_async_copy(
        out_v, out_hbm.at[pl.ds(base * K, ROWS_PER_W * K)], sem_a)
    out_copy.start()
    out_copy.wait()


@jax.jit
def kernel(x):
    B, S, _ = x.shape
    mesh = plsc.VectorSubcoreMesh(core_axis_name="c", subcore_axis_name="s")
    run = pl.kernel(
        _sc_topk,
        out_type=jax.ShapeDtypeStruct((R_TOTAL * K,), jnp.float32),
        mesh=mesh,
        compiler_params=pltpu.CompilerParams(needs_layout_passes=False),
        scratch_types=[
            pltpu.VMEM((BATCH * N,), jnp.float32),
            pltpu.VMEM((BATCH * N,), jnp.float32),
            pltpu.VMEM((ROWS_PER_W * K,), jnp.float32),
            pltpu.SemaphoreType.DMA,
            pltpu.SemaphoreType.DMA,
        ],
    )
    out = run(x)
    return out.reshape(B, S, K)


# speedup vs baseline: 1.3805x; 1.3805x over previous
import functools

import jax
import jax.numpy as jnp
from jax import lax
from jax.experimental import pallas as pl
from jax.experimental.pallas import tpu as pltpu
from jax.experimental.pallas import tpu_sc as plsc

K = 64
N = 8192
L = 16
R_TOTAL = 8192
NW = 32
ROWS_PER_W = R_TOTAL // NW
BATCH = 8
NGRP = 32


def _rev(v):
    return lax.rev(v, (0,))


def _sort16(v, desc):
    if desc:
        return plsc.sort_key_val(v, v, descending=True)[0]
    return jnp.sort(v)


def _sort_bitonic(vs, desc):
    if len(vs) == 1:
        return [_sort16(vs[0], desc)]
    h = len(vs) // 2
    lo = [jnp.minimum(a, b) for a, b in zip(vs[:h], vs[h:])]
    hi = [jnp.maximum(a, b) for a, b in zip(vs[:h], vs[h:])]
    if desc:
        return _sort_bitonic(hi, True) + _sort_bitonic(lo, True)
    return _sort_bitonic(lo, False) + _sort_bitonic(hi, False)


def _merge(A, B, desc, cap=False):
    hi = [jnp.maximum(a, b) for a, b in zip(A, B)]
    if cap:
        return _sort_bitonic(hi, desc)
    lo = [jnp.minimum(a, b) for a, b in zip(A, B)]
    if desc:
        return _sort_bitonic(hi, True) + _sort_bitonic(lo, True)
    return _sort_bitonic(lo, False) + _sort_bitonic(hi, False)


def _kv_sort_bitonic(ks, vs, desc):
    if len(ks) == 1:
        sk, sv = plsc.sort_key_val(ks[0], vs[0], descending=desc)
        return [sk], [sv]
    h = len(ks) // 2
    m = [a <= b for a, b in zip(ks[:h], ks[h:])]
    lok = [jnp.minimum(a, b) for a, b in zip(ks[:h], ks[h:])]
    hik = [jnp.maximum(a, b) for a, b in zip(ks[:h], ks[h:])]
    lov = [jnp.where(mm, a, b) for mm, a, b in zip(m, vs[:h], vs[h:])]
    hiv = [jnp.where(mm, b, a) for mm, a, b in zip(m, vs[:h], vs[h:])]
    if desc:
        k1, v1 = _kv_sort_bitonic(hik, hiv, True)
        k2, v2 = _kv_sort_bitonic(lok, lov, True)
        return k1 + k2, v1 + v2
    k1, v1 = _kv_sort_bitonic(lok, lov, False)
    k2, v2 = _kv_sort_bitonic(hik, hiv, False)
    return k1 + k2, v1 + v2


def _kv_merge(Ak, Av, Bk, Bv, desc, cap=False):
    m = [a <= b for a, b in zip(Ak, Bk)]
    hik = [jnp.maximum(a, b) for a, b in zip(Ak, Bk)]
    hiv = [jnp.where(mm, b, a) for mm, a, b in zip(m, Av, Bv)]
    if cap:
        return _kv_sort_bitonic(hik, hiv, desc)
    lok = [jnp.minimum(a, b) for a, b in zip(Ak, Bk)]
    lov = [jnp.where(mm, a, b) for mm, a, b in zip(m, Av, Bv)]
    if desc:
        k1, v1 = _kv_sort_bitonic(hik, hiv, True)
        k2, v2 = _kv_sort_bitonic(lok, lov, True)
        return k1 + k2, v1 + v2
    k1, v1 = _kv_sort_bitonic(lok, lov, False)
    k2, v2 = _kv_sort_bitonic(hik, hiv, False)
    return k1 + k2, v1 + v2


def _process_row(row_v, out_v, off, r):
    iota = lax.iota(jnp.int32, L)
    lists = []
    for g in range(NGRP):
        vs = [row_v[pl.ds(off + g * 256 + j * L, L)] for j in range(16)]
        while len(vs) > 1:
            vs = [jnp.maximum(a, b) for a, b in zip(vs[0::2], vs[1::2])]
        sk, sv = plsc.sort_key_val(vs[0], g * 256 + iota,
                                   descending=(g % 2 == 1))
        lists.append(([sk], [sv]))
    while len(lists) > 1:
        nxt = []
        for i, ((ak, av), (bk, bv)) in enumerate(
                zip(lists[0::2], lists[1::2])):
            nxt.append(_kv_merge(ak, av, bk, bv, desc=(i % 2 == 1),
                                 cap=(len(ak) == 4)))
        lists = nxt
    vals4 = lists[0][1]
    leaves = []
    for idx, v in enumerate(vals4):
        base = v + off
        for j in range(16):
            i = idx * 16 + j
            leaves.append(_sort16(plsc.load_gather(row_v, [base + j * L]),
                                  desc=(i % 2 == 1)))
    ls = [[x] for x in leaves]
    while len(ls) > 1:
        ls = [_merge(a, b, desc=(i % 2 == 1), cap=(len(a) == 4))
              for i, (a, b) in enumerate(zip(ls[0::2], ls[1::2]))]
    top = ls[0]
    for j in range(4):
        out_v[pl.ds(r * K + j * L, L)] = _rev(top[3 - j])


def _sc_topk(x_hbm, out_hbm, row_v, out_v, sem):
    wid = lax.axis_index("s") * 2 + lax.axis_index("c")
    base = wid * ROWS_PER_W

    def batch_body(b, _):
        rows0 = base + b * BATCH
        bb = rows0 // 1024
        ss = rows0 % 1024
        copies = [
            pltpu.make_async_copy(
                x_hbm.at[bb, ss + i], row_v.at[pl.ds(i * N, N)], sem)
            for i in range(BATCH)
        ]
        for c in copies:
            c.start()
        for c in copies:
            c.wait()

        def row_body(i, _):
            _process_row(row_v, out_v, i * N, b * BATCH + i)
            return 0

        lax.fori_loop(0, BATCH, row_body, 0, unroll=False)
        return 0

    lax.fori_loop(0, ROWS_PER_W // BATCH, batch_body, 0, unroll=False)

    out_copy = pltpu.make_async_copy(
        out_v, out_hbm.at[pl.ds(base * K, ROWS_PER_W * K)], sem)
    out_copy.start()
    out_copy.wait()


@jax.jit
def kernel(x):
    B, S, _ = x.shape
    mesh = plsc.VectorSubcoreMesh(core_axis_name="c", subcore_axis_name="s")
    run = pl.kernel(
        _sc_topk,
        out_type=jax.ShapeDtypeStruct((R_TOTAL * K,), jnp.float32),
        mesh=mesh,
        compiler_params=pltpu.CompilerParams(needs_layout_passes=False),
        scratch_types=[
            pltpu.VMEM((BATCH * N,), jnp.float32),
            pltpu.VMEM((ROWS_PER_W * K,), jnp.float32),
            pltpu.SemaphoreType.DMA,
        ],
    )
    out = run(x)
    return out.reshape(B, S, K)
